# baseline (speedup 1.0000x reference)
import functools

import jax
import jax.numpy as jnp
from jax import lax
from jax.experimental import pallas as pl
from jax.experimental.pallas import tpu as pltpu
from jax.experimental.pallas import tpu_sc as plsc

B = 16384
D = 64
CTX = 20
NEG = 20
V = 1000000

NC = 2
NS = 16
NW = NC * NS
LANES = 16

BPW = B // NW
CB = 32
NCHUNK = BPW // CB
GROUPS = CB * CTX // 128
SCOL = 32


def _sc_scores(ctx_ids, pos_ids, neg_ids, in_w2, out_w2):
  mesh = plsc.VectorSubcoreMesh(core_axis_name="c", subcore_axis_name="s")

  @functools.partial(
      pl.kernel,
      mesh=mesh,
      out_type=jax.ShapeDtypeStruct((B, SCOL), jnp.float32),
      scratch_types=[
          pltpu.VMEM((CB * CTX,), jnp.int32),
          pltpu.VMEM((CB * NEG,), jnp.int32),
          pltpu.VMEM((CB + LANES,), jnp.int32),
          pltpu.VMEM((CB,), jnp.int32),
          pltpu.VMEM((CB * CTX,), jnp.int32),
          pltpu.VMEM((CB * CTX, 128), jnp.float32),
          pltpu.VMEM((CB, 128), jnp.float32),
          pltpu.VMEM((CB, D), jnp.float32),
          pltpu.VMEM((CB, SCOL), jnp.float32),
          pltpu.SemaphoreType.DMA,
      ],
  )
  def k(ctx_ids_hbm, pos_hbm, neg_ids_hbm, in_hbm, out_hbm, scores_hbm,
        ctx_ids_v, neg_ids_v, pos_ids_v, pos_pair_v, pair_v, rows_v,
        pos_rows, means_v, scores_v, sem):
    wid = lax.axis_index("s") * NC + lax.axis_index("c")
    lane = lax.iota(jnp.int32, LANES)
    rot = [(lane + sh) & (LANES - 1) for sh in (8, 4, 2, 1)]
    dnums = lax.GatherDimensionNumbers(
        offset_dims=(), collapsed_slice_dims=(0,), start_index_map=(0,))

    def hsum(v):
      for r in rot:
        v = v + lax.gather(
            v, r[:, None], dimension_numbers=dnums, slice_sizes=(1,),
            mode=lax.GatherScatterMode.PROMISE_IN_BOUNDS)
      return v

    H = V // 2

    def derive_pairs(src_v, n):
      for i in range(n // LANES):
        sl = pl.ds(i * LANES, LANES)
        v = src_v[sl]
        pair_v[sl] = jnp.where(v >= H, v - H, v)

    def fire_gathers(table, n):
      copies = []
      for g in range(n // 128):
        copies.append(pltpu.async_copy(
            table.at[pair_v.at[pl.ds(g * 128, 128)]],
            rows_v.at[pl.ds(g * 128, 128)], sem))
      return copies

    def load_half(ref, r, src_id, q):
      off = jnp.where(src_id >= H, D, 0) + q * LANES
      return ref[r, pl.ds(off, LANES)]

    def chunk_body(chunk, _):
      b0 = wid * BPW + chunk * CB
      pltpu.sync_copy(ctx_ids_hbm.at[pl.ds(b0 * CTX, CB * CTX)], ctx_ids_v)
      pltpu.sync_copy(neg_ids_hbm.at[pl.ds(b0 * NEG, CB * NEG)], neg_ids_v)
      pltpu.sync_copy(pos_hbm.at[pl.ds(b0, CB)], pos_ids_v.at[pl.ds(0, CB)])

      derive_pairs(ctx_ids_v, CB * CTX)
      for cp in fire_gathers(in_hbm, CB * CTX):
        cp.wait()

      def mean_body(b, _):
        base = b * CTX
        ids_a = ctx_ids_v[pl.ds(base, LANES)]
        ids_b = ctx_ids_v[pl.ds(base + CTX - LANES, LANES)]

        def cid(c):
          return ids_a[c] if c < LANES else ids_b[c - (CTX - LANES)]

        acc = [load_half(rows_v, base, cid(0), q) for q in range(4)]
        for c in range(1, CTX):
          for q in range(4):
            acc[q] = acc[q] + load_half(rows_v, base + c, cid(c), q)
        scale = jnp.float32(1.0 / CTX)
        for q in range(4):
          means_v[b, pl.ds(q * LANES, LANES)] = acc[q] * scale
        return 0

      lax.fori_loop(0, CB, mean_body, 0)

      derive_pairs(neg_ids_v, CB * NEG)
      copies = fire_gathers(out_hbm, CB * NEG)
      for i in range(CB // LANES):
        sl = pl.ds(i * LANES, LANES)
        v = pos_ids_v[sl]
        pos_pair_v[sl] = jnp.where(v >= H, v - H, v)
      copies.append(pltpu.async_copy(out_hbm.at[pos_pair_v], pos_rows, sem))
      for cp in copies:
        cp.wait()

      def dot_body(b, _):
        mean = [means_v[b, pl.ds(q * LANES, LANES)] for q in range(4)]
        nids_a = neg_ids_v[pl.ds(b * NEG, LANES)]
        nids_b = neg_ids_v[pl.ds(b * NEG + NEG - LANES, LANES)]
        pid = pos_ids_v[pl.ds(b, LANES)][0]

        def nid(n):
          return nids_a[n] if n < LANES else nids_b[n - (NEG - LANES)]

        def dot_row(ref, r, src_id):
          p = mean[0] * load_half(ref, r, src_id, 0)
          for q in range(1, 4):
            p = p + mean[q] * load_half(ref, r, src_id, q)
          return hsum(p)

        s_lo = jnp.zeros((LANES,), jnp.float32)
        s_hi = jnp.zeros((LANES,), jnp.float32)
        s_lo = jnp.where(lane == 0, dot_row(pos_rows, b, pid), s_lo)
        for n in range(NEG):
          j = 1 + n
          s = dot_row(rows_v, b * NEG + n, nid(n))
          if j < LANES:
            s_lo = jnp.where(lane == j, s, s_lo)
          else:
            s_hi = jnp.where(lane == j - LANES, s, s_hi)
        scores_v[b, pl.ds(0, LANES)] = s_lo
        scores_v[b, pl.ds(LANES, LANES)] = s_hi
        return 0

      lax.fori_loop(0, CB, dot_body, 0)
      pltpu.sync_copy(scores_v, scores_hbm.at[pl.ds(b0, CB)])
      return 0

    lax.fori_loop(0, NCHUNK, chunk_body, 0)

  return k(ctx_ids, pos_ids, neg_ids, in_w2, out_w2)


def _sc_repack(w):
  H = V // 2
  RR = 128
  NCHR = H // RR
  NFULL = NCHR // NW
  NREM = NCHR - NFULL * NW
  TAIL = H - NCHR * RR
  mesh = plsc.VectorSubcoreMesh(core_axis_name="c", subcore_axis_name="s")

  @functools.partial(
      pl.kernel,
      mesh=mesh,
      out_type=jax.ShapeDtypeStruct((H, 2 * D), jnp.float32),
      compiler_params=pltpu.CompilerParams(use_tc_tiling_on_sc=True),
      scratch_types=[
          pltpu.VMEM((RR, D), jnp.float32),
          pltpu.VMEM((RR, D), jnp.float32),
          pltpu.VMEM((RR, 2 * D), jnp.float32),
          pltpu.SemaphoreType.DMA,
          pltpu.SemaphoreType.DMA,
      ],
  )
  def k(w_hbm, o_hbm, a, b, c, rsem, wsem):
    wid = lax.axis_index("s") * NC + lax.axis_index("c")

    def interleave(n):
      def rows(i, _):
        for r in (2 * i, 2 * i + 1):
          for q in range(4):
            sl = pl.ds(q * LANES, LANES)
            c[r, sl] = a[r, sl]
            c[r, pl.ds(D + q * LANES, LANES)] = b[r, sl]
        return 0

      lax.fori_loop(0, n // 2, rows, 0)

    def do_chunk(k0, t):
      ra = pltpu.async_copy(w_hbm.at[pl.ds(k0, RR)], a, rsem)
      rb = pltpu.async_copy(w_hbm.at[pl.ds(H + k0, RR)], b, rsem)

      @pl.when(t > 0)
      def _():
        pltpu.make_async_copy(o_hbm.at[pl.ds(0, RR)], c, wsem).wait()

      ra.wait()
      rb.wait()
      interleave(RR)
      pltpu.async_copy(c, o_hbm.at[pl.ds(k0, RR)], wsem)

    def step(t, _):
      do_chunk((t * NW + wid) * RR, t)
      return 0

    lax.fori_loop(0, NFULL, step, 0)
    pltpu.make_async_copy(o_hbm.at[pl.ds(0, RR)], c, wsem).wait()

    @pl.when(wid < NREM)
    def _():
      k0 = (NFULL * NW + wid) * RR
      pltpu.sync_copy(w_hbm.at[pl.ds(k0, RR)], a)
      pltpu.sync_copy(w_hbm.at[pl.ds(H + k0, RR)], b)
      interleave(RR)
      pltpu.sync_copy(c, o_hbm.at[pl.ds(k0, RR)])

    @pl.when(wid == NREM)
    def _():
      k0 = NCHR * RR
      pltpu.sync_copy(w_hbm.at[pl.ds(k0, TAIL)], a.at[pl.ds(0, TAIL)])
      pltpu.sync_copy(w_hbm.at[pl.ds(H + k0, TAIL)], b.at[pl.ds(0, TAIL)])
      interleave(TAIL)
      pltpu.sync_copy(c.at[pl.ds(0, TAIL)], o_hbm.at[pl.ds(k0, TAIL)])

  return k(w)


def _tc_loss(scores):
  rows = B * SCOL // 128

  def body(s_ref, o_ref):
    s = s_ref[...]
    col = lax.broadcasted_iota(jnp.int32, s.shape, 1) % SCOL
    def sp(x):
      return jnp.maximum(x, 0.0) + jnp.log(1.0 + jnp.exp(-jnp.abs(x)))
    contrib = jnp.where(col == 0, sp(-s), 0.0)
    contrib = contrib + jnp.where((col >= 1) & (col <= NEG), sp(s), 0.0)
    o_ref[0, 0] = jnp.sum(contrib) * jnp.float32(1.0 / B)

  out = pl.pallas_call(
      body,
      out_shape=jax.ShapeDtypeStruct((1, 1), jnp.float32),
      in_specs=[pl.BlockSpec((rows, 128), lambda: (0, 0))],
      out_specs=pl.BlockSpec((1, 1), lambda: (0, 0),
                             memory_space=pltpu.SMEM),
  )(scores.reshape(rows, 128))
  return out[0, 0]


def kernel(context_ids, pos_ids, neg_ids, in_embed_weight, out_embed_weight):
  in_w2 = _sc_repack(in_embed_weight)
  out_w2 = _sc_repack(out_embed_weight)
  ctx_idx = context_ids.reshape(B * CTX)
  neg_idx = neg_ids.reshape(B * NEG)
  scores = _sc_scores(ctx_idx, pos_ids, neg_idx, in_w2, out_w2)
  return _tc_loss(scores)


# speedup vs baseline: 1.1740x; 1.1740x over previous
import functools

import jax
import jax.numpy as jnp
from jax import lax
from jax.experimental import pallas as pl
from jax.experimental.pallas import tpu as pltpu
from jax.experimental.pallas import tpu_sc as plsc

B = 16384
D = 64
CTX = 20
NEG = 20
V = 1000000

NC = 2
NS = 16
NW = NC * NS
LANES = 16

BPW = B // NW
CB = 32
NCHUNK = BPW // CB
GROUPS = CB * CTX // 128
SCOL = 32


def _sc_scores(ctx_idx, pos_ids, neg_idx, in_w, out_w):
  mesh = plsc.VectorSubcoreMesh(core_axis_name="c", subcore_axis_name="s")

  @functools.partial(
      pl.kernel,
      mesh=mesh,
      out_type=jax.ShapeDtypeStruct((B, SCOL), jnp.float32),
      compiler_params=pltpu.CompilerParams(use_tc_tiling_on_sc=False),
      scratch_types=[
          pltpu.VMEM((CB * CTX,), jnp.int32),
          pltpu.VMEM((CB * NEG,), jnp.int32),
          pltpu.VMEM((CB,), jnp.int32),
          pltpu.VMEM((CB * CTX, D), jnp.float32),
          pltpu.VMEM((CB * NEG, D), jnp.float32),
          pltpu.VMEM((CB, D), jnp.float32),
          pltpu.VMEM((CB, SCOL), jnp.float32),
          pltpu.SemaphoreType.DMA,
      ],
  )
  def k(ctx_idx_hbm, pos_hbm, neg_idx_hbm, in_hbm, out_hbm, scores_hbm,
        ctx_idx_v, neg_idx_v, pos_idx_v, ctx_rows, neg_rows, pos_rows,
        scores_v, sem):
    wid = lax.axis_index("s") * NC + lax.axis_index("c")

    def chunk_body(chunk, _):
      b0 = wid * BPW + chunk * CB
      pltpu.sync_copy(ctx_idx_hbm.at[pl.ds(b0 * CTX, CB * CTX)], ctx_idx_v)
      pltpu.sync_copy(neg_idx_hbm.at[pl.ds(b0 * NEG, CB * NEG)], neg_idx_v)
      pltpu.sync_copy(pos_hbm.at[pl.ds(b0, CB)], pos_idx_v)
      copies = []
      for g in range(GROUPS):
        copies.append(pltpu.async_copy(
            in_hbm.at[ctx_idx_v.at[pl.ds(g * 128, 128)]],
            ctx_rows.at[pl.ds(g * 128, 128)], sem))
      for g in range(GROUPS):
        copies.append(pltpu.async_copy(
            out_hbm.at[neg_idx_v.at[pl.ds(g * 128, 128)]],
            neg_rows.at[pl.ds(g * 128, 128)], sem))
      copies.append(pltpu.async_copy(out_hbm.at[pos_idx_v], pos_rows, sem))
      for cp in copies:
        cp.wait()

      lane = lax.iota(jnp.int32, LANES)
      rot = [(lane + sh) & (LANES - 1) for sh in (8, 4, 2, 1)]
      dnums = lax.GatherDimensionNumbers(
          offset_dims=(), collapsed_slice_dims=(0,), start_index_map=(0,))

      def hsum(v):
        for r in rot:
          v = v + lax.gather(
              v, r[:, None], dimension_numbers=dnums, slice_sizes=(1,),
              mode=lax.GatherScatterMode.PROMISE_IN_BOUNDS)
        return v

      def batch_body(b, _):
        base = b * CTX
        mean = [ctx_rows[base, pl.ds(q * LANES, LANES)] for q in range(4)]
        for c in range(1, CTX):
          for q in range(4):
            mean[q] = mean[q] + ctx_rows[base + c, pl.ds(q * LANES, LANES)]
        scale = jnp.float32(1.0 / CTX)
        mean = [m * scale for m in mean]

        def dot_row(row_ref, r):
          p = mean[0] * row_ref[r, pl.ds(0, LANES)]
          for q in range(1, 4):
            p = p + mean[q] * row_ref[r, pl.ds(q * LANES, LANES)]
          return hsum(p)

        s_lo = jnp.zeros((LANES,), jnp.float32)
        s_hi = jnp.zeros((LANES,), jnp.float32)
        s_lo = jnp.where(lane == 0, dot_row(pos_rows, b), s_lo)
        for n in range(NEG):
          j = 1 + n
          s = dot_row(neg_rows, b * NEG + n)
          if j < LANES:
            s_lo = jnp.where(lane == j, s, s_lo)
          else:
            s_hi = jnp.where(lane == j - LANES, s, s_hi)
        scores_v[b, pl.ds(0, LANES)] = s_lo
        scores_v[b, pl.ds(LANES, LANES)] = s_hi
        return 0

      lax.fori_loop(0, CB, batch_body, 0)
      pltpu.sync_copy(scores_v, scores_hbm.at[pl.ds(b0, CB)])
      return 0

    lax.fori_loop(0, NCHUNK, chunk_body, 0)

  return k(ctx_idx, pos_ids, neg_idx, in_w, out_w)


def _tc_loss(scores):
  rows = B * SCOL // 128

  def body(s_ref, o_ref):
    s = s_ref[...]
    col = lax.broadcasted_iota(jnp.int32, s.shape, 1) % SCOL
    def sp(x):
      return jnp.maximum(x, 0.0) + jnp.log(1.0 + jnp.exp(-jnp.abs(x)))
    contrib = jnp.where(col == 0, sp(-s), 0.0)
    contrib = contrib + jnp.where((col >= 1) & (col <= NEG), sp(s), 0.0)
    o_ref[0, 0] = jnp.sum(contrib) * jnp.float32(1.0 / B)

  out = pl.pallas_call(
      body,
      out_shape=jax.ShapeDtypeStruct((1, 1), jnp.float32),
      in_specs=[pl.BlockSpec((rows, 128), lambda: (0, 0))],
      out_specs=pl.BlockSpec((1, 1), lambda: (0, 0),
                             memory_space=pltpu.SMEM),
  )(scores.reshape(rows, 128))
  return out[0, 0]


def kernel(context_ids, pos_ids, neg_ids, in_embed_weight, out_embed_weight):
  ctx_idx = context_ids.reshape(B * CTX)
  neg_idx = neg_ids.reshape(B * NEG)
  scores = _sc_scores(ctx_idx, pos_ids, neg_idx,
                      in_embed_weight, out_embed_weight)
  return _tc_loss(scores)
